# baseline (speedup 1.0000x reference)
import functools

import jax
import jax.numpy as jnp
from jax import lax
from jax.experimental import pallas as pl
from jax.experimental.pallas import tpu as pltpu
from jax.experimental.pallas import tpu_sc as plsc

NUM_FACTORS = 16
NUM_ROWS = 1000000
BATCH = 16384
_ROWS_PER_BLOCK = 8
_NBLK = NUM_ROWS // _ROWS_PER_BLOCK

_NC, _NS = 2, 16
_NW = _NC * _NS
_BPW = BATCH // _NW
_CH = 128
_NCH = _BPW // _CH
_GROUP = 16


def _mf_body(user_hbm, item_hbm, uf_hbm, if_hbm, out_hbm,
             uidx_v, iidx_v, ublk_v, iblk_v, out_v,
             ub, vb, sems):
    wid = lax.axis_index("s") * _NC + lax.axis_index("c")
    base = wid * _BPW

    pltpu.sync_copy(user_hbm.at[pl.ds(base, _BPW)], uidx_v)
    pltpu.sync_copy(item_hbm.at[pl.ds(base, _BPW)], iidx_v)

    def blkstep(g, carry):
        o = g * _GROUP
        ublk_v[pl.ds(o, _GROUP)] = lax.shift_right_logical(
            uidx_v[pl.ds(o, _GROUP)], 3)
        iblk_v[pl.ds(o, _GROUP)] = lax.shift_right_logical(
            iidx_v[pl.ds(o, _GROUP)], 3)
        return carry

    lax.fori_loop(0, _BPW // _GROUP, blkstep, 0, unroll=False)

    def issue(c):
        buf = c % 2
        cu = pltpu.async_copy(
            uf_hbm.at[ublk_v.at[pl.ds(c * _CH, _CH)]], ub.at[buf], sems.at[2 * buf])
        cv = pltpu.async_copy(
            if_hbm.at[iblk_v.at[pl.ds(c * _CH, _CH)]], vb.at[buf], sems.at[2 * buf + 1])
        return cu, cv

    lane = lax.iota(jnp.int32, 16)
    diags = [(lane + k) & 15 for k in range(16)]

    def compute_chunk(c):
        buf = c % 2
        c0 = c * _CH

        def step(g, carry):
            o = g * _GROUP
            rows = lane + o
            usub = lax.shift_left(uidx_v[pl.ds(c0 + o, _GROUP)] & 7, 4)
            isub = lax.shift_left(iidx_v[pl.ds(c0 + o, _GROUP)] & 7, 4)
            acc = jnp.zeros((16,), jnp.float32)
            for k in range(16):
                du = plsc.load_gather(ub.at[buf], [rows, usub + diags[k]])
                dv = plsc.load_gather(vb.at[buf], [rows, isub + diags[k]])
                acc = acc + du * dv
            out_v[pl.ds(c0 + o, _GROUP)] = acc
            return carry

        lax.fori_loop(0, _CH // _GROUP, step, 0, unroll=False)

    cps = issue(0)
    for c in range(_NCH):
        cps[0].wait()
        cps[1].wait()
        if c + 1 < _NCH:
            cps = issue(c + 1)
        compute_chunk(c)

    pltpu.sync_copy(out_v, out_hbm.at[pl.ds(base, _BPW)])


@jax.jit
def _mf_call(user, item, user_factors, item_factors):
    eye = jnp.eye(NUM_FACTORS, dtype=jnp.float32)
    uf_blocks = lax.dot(
        user_factors, eye, precision=lax.Precision.HIGHEST
    ).reshape(_NBLK, 128)
    if_blocks = lax.dot(
        item_factors, eye, precision=lax.Precision.HIGHEST
    ).reshape(_NBLK, 128)
    mesh = plsc.VectorSubcoreMesh(
        core_axis_name="c", subcore_axis_name="s",
        num_cores=_NC, num_subcores=_NS)
    return pl.kernel(
        _mf_body,
        out_type=jax.ShapeDtypeStruct((BATCH,), jnp.float32),
        mesh=mesh,
        compiler_params=pltpu.CompilerParams(
            needs_layout_passes=False, use_tc_tiling_on_sc=True),
        scratch_types=[
            pltpu.VMEM((_BPW,), jnp.int32),
            pltpu.VMEM((_BPW,), jnp.int32),
            pltpu.VMEM((_BPW,), jnp.int32),
            pltpu.VMEM((_BPW,), jnp.int32),
            pltpu.VMEM((_BPW,), jnp.float32),
            pltpu.VMEM((2, _CH, 128), jnp.float32),
            pltpu.VMEM((2, _CH, 128), jnp.float32),
            pltpu.SemaphoreType.DMA((4,)),
        ],
    )(user, item, uf_blocks, if_blocks)


def kernel(user, item, user_factors, item_factors):
    user = user.astype(jnp.int32)
    item = item.astype(jnp.int32)
    return _mf_call(user, item, user_factors, item_factors)


# speedup vs baseline: 5.2483x; 5.2483x over previous
import functools

import jax
import jax.numpy as jnp
from jax import lax
from jax.experimental import pallas as pl
from jax.experimental.pallas import tpu as pltpu
from jax.experimental.pallas import tpu_sc as plsc

NUM_FACTORS = 16
NUM_ROWS = 1000000
BATCH = 16384

_NC, _NS = 2, 16
_NW = _NC * _NS
_BPW = BATCH // _NW
_CHUNK = 16
_NCHUNK = _BPW // _CHUNK


def _mf_body(user_hbm, item_hbm, tu_hbm, tv_hbm, out_hbm,
             uidx_v, iidx_v, rows_u, rows_v, out_v, tbuf, sem):
    wid = lax.axis_index("s") * _NC + lax.axis_index("c")
    base = wid * _BPW

    pltpu.sync_copy(user_hbm.at[pl.ds(base, _BPW)], uidx_v)
    pltpu.sync_copy(item_hbm.at[pl.ds(base, _BPW)], iidx_v)

    lane = lax.iota(jnp.int32, 16)

    def gather_table(tbl_hbm, idx_v, rows_out):
        def chunk(c, carry):
            o = c * _CHUNK
            idx16 = idx_v[pl.ds(o, _CHUNK)]
            tcol = lax.shift_right_logical(idx16, 7)
            m16 = idx16 & 127
            cps = []
            for j in range(_CHUNK):
                col0 = pl.multiple_of(tcol[j] * 128, 128)
                cps.append(pltpu.async_copy(
                    tbl_hbm.at[:, pl.ds(col0, 128)],
                    tbuf.at[pl.ds(j * 16, 16), :], sem))
            for cp in cps:
                cp.wait()
            for j in range(_CHUNK):
                mj = jnp.full((16,), m16[j], dtype=jnp.int32)
                col = plsc.load_gather(tbuf, [j * 16 + lane, mj])
                rows_out[pl.ds((o + j) * 16, 16)] = col
            return carry

        lax.fori_loop(0, _NCHUNK, chunk, 0, unroll=False)

    gather_table(tu_hbm, uidx_v, rows_u)
    gather_table(tv_hbm, iidx_v, rows_v)

    diags = [(lane + k) & 15 for k in range(16)]

    def step(g, carry):
        r0 = g * 16
        rows = r0 + lane
        acc = jnp.zeros((16,), jnp.float32)
        for k in range(16):
            flat = rows * 16 + diags[k]
            du = plsc.load_gather(rows_u, [flat])
            dv = plsc.load_gather(rows_v, [flat])
            acc = acc + du * dv
        out_v[pl.ds(r0, 16)] = acc
        return carry

    lax.fori_loop(0, _BPW // 16, step, 0, unroll=False)

    pltpu.sync_copy(out_v, out_hbm.at[pl.ds(base, _BPW)])


@jax.jit
def _mf_call(user, item, tu, tv):
    mesh = plsc.VectorSubcoreMesh(
        core_axis_name="c", subcore_axis_name="s",
        num_cores=_NC, num_subcores=_NS)
    return pl.kernel(
        _mf_body,
        out_type=jax.ShapeDtypeStruct((BATCH,), jnp.float32),
        mesh=mesh,
        compiler_params=pltpu.CompilerParams(
            needs_layout_passes=False, use_tc_tiling_on_sc=True),
        scratch_types=[
            pltpu.VMEM((_BPW,), jnp.int32),
            pltpu.VMEM((_BPW,), jnp.int32),
            pltpu.VMEM((_BPW * NUM_FACTORS,), jnp.float32),
            pltpu.VMEM((_BPW * NUM_FACTORS,), jnp.float32),
            pltpu.VMEM((_BPW,), jnp.float32),
            pltpu.VMEM((_CHUNK * 16, 128), jnp.float32),
            pltpu.SemaphoreType.DMA,
        ],
    )(user, item, tu, tv)


def kernel(user, item, user_factors, item_factors):
    user = user.astype(jnp.int32)
    item = item.astype(jnp.int32)
    return _mf_call(user, item, user_factors.T, item_factors.T)


# speedup vs baseline: 6.4079x; 1.2209x over previous
import functools

import jax
import jax.numpy as jnp
from jax import lax
from jax.experimental import pallas as pl
from jax.experimental.pallas import tpu as pltpu
from jax.experimental.pallas import tpu_sc as plsc

NUM_FACTORS = 16
NUM_ROWS = 1000000
BATCH = 16384

_NC, _NS = 2, 16
_NW = _NC * _NS
_BPW = BATCH // _NW
_CHUNK = 16
_NCHUNK = _BPW // _CHUNK


def _mf_body(user_hbm, item_hbm, tu_hbm, tv_hbm, out_hbm,
             uidx_v, iidx_v, rows_u, rows_v, out_v, tbuf_a, tbuf_b, sem_a, sem_b):
    wid = lax.axis_index("s") * _NC + lax.axis_index("c")
    base = wid * _BPW

    pltpu.sync_copy(user_hbm.at[pl.ds(base, _BPW)], uidx_v)
    pltpu.sync_copy(item_hbm.at[pl.ds(base, _BPW)], iidx_v)

    lane = lax.iota(jnp.int32, 16)

    def gather_table(tbl_hbm, idx_v, rows_out, tbuf, sem, tbuf2, sem2):
        def issue(c, buf, s):
            o = c * _CHUNK
            tcol = lax.shift_right_logical(idx_v[pl.ds(o, _CHUNK)], 7)
            for j in range(_CHUNK):
                col0 = pl.multiple_of(tcol[j] * 128, 128)
                pltpu.async_copy(
                    tbl_hbm.at[:, pl.ds(col0, 128)],
                    buf.at[pl.ds(j * 16, 16), :], s)

        def drain(buf, s):
            pltpu.make_async_copy(
                tbl_hbm.at[:, pl.ds(0, _CHUNK * 16)], buf, s).wait()

        def extract(c, buf):
            o = c * _CHUNK
            m16 = idx_v[pl.ds(o, _CHUNK)] & 127
            for j in range(_CHUNK):
                mj = jnp.full((16,), m16[j], dtype=jnp.int32)
                col = plsc.load_gather(buf, [j * 16 + lane, mj])
                rows_out[pl.ds((o + j) * 16, 16)] = col

        def pair(g, carry):
            c0 = g * 2
            issue(c0, tbuf, sem)

            @pl.when(g > 0)
            def _():
                drain(tbuf2, sem2)
                extract(c0 - 1, tbuf2)

            issue(c0 + 1, tbuf2, sem2)
            drain(tbuf, sem)
            extract(c0, tbuf)
            return carry

        lax.fori_loop(0, _NCHUNK // 2, pair, 0, unroll=False)
        drain(tbuf2, sem2)
        extract(_NCHUNK - 1, tbuf2)

    gather_table(tu_hbm, uidx_v, rows_u, tbuf_a, sem_a, tbuf_b, sem_b)
    gather_table(tv_hbm, iidx_v, rows_v, tbuf_a, sem_a, tbuf_b, sem_b)

    diags = [(lane + k) & 15 for k in range(16)]

    def step(g, carry):
        r0 = g * 16
        rows = r0 + lane
        acc = jnp.zeros((16,), jnp.float32)
        for k in range(16):
            flat = rows * 16 + diags[k]
            du = plsc.load_gather(rows_u, [flat])
            dv = plsc.load_gather(rows_v, [flat])
            acc = acc + du * dv
        out_v[pl.ds(r0, 16)] = acc
        return carry

    lax.fori_loop(0, _BPW // 16, step, 0, unroll=False)

    pltpu.sync_copy(out_v, out_hbm.at[pl.ds(base, _BPW)])


@jax.jit
def _mf_call(user, item, tu, tv):
    mesh = plsc.VectorSubcoreMesh(
        core_axis_name="c", subcore_axis_name="s",
        num_cores=_NC, num_subcores=_NS)
    return pl.kernel(
        _mf_body,
        out_type=jax.ShapeDtypeStruct((BATCH,), jnp.float32),
        mesh=mesh,
        compiler_params=pltpu.CompilerParams(
            needs_layout_passes=False, use_tc_tiling_on_sc=True),
        scratch_types=[
            pltpu.VMEM((_BPW,), jnp.int32),
            pltpu.VMEM((_BPW,), jnp.int32),
            pltpu.VMEM((_BPW * NUM_FACTORS,), jnp.float32),
            pltpu.VMEM((_BPW * NUM_FACTORS,), jnp.float32),
            pltpu.VMEM((_BPW,), jnp.float32),
            pltpu.VMEM((_CHUNK * 16, 128), jnp.float32),
            pltpu.VMEM((_CHUNK * 16, 128), jnp.float32),
            pltpu.SemaphoreType.DMA,
            pltpu.SemaphoreType.DMA,
        ],
    )(user, item, tu, tv)


def kernel(user, item, user_factors, item_factors):
    user = user.astype(jnp.int32)
    item = item.astype(jnp.int32)
    return _mf_call(user, item, user_factors.T, item_factors.T)


# speedup vs baseline: 6.5062x; 1.0153x over previous
import functools

import jax
import jax.numpy as jnp
from jax import lax
from jax.experimental import pallas as pl
from jax.experimental.pallas import tpu as pltpu
from jax.experimental.pallas import tpu_sc as plsc

NUM_FACTORS = 16
NUM_ROWS = 1000000
BATCH = 16384

_NC, _NS = 2, 16
_NW = _NC * _NS
_BPW = BATCH // _NW
_CHUNK = 16
_NCHUNK = _BPW // _CHUNK


def _mf_body(user_hbm, item_hbm, tu_hbm, tv_hbm, out_hbm,
             uidx_v, iidx_v, rows_u, rows_v, out_v, tbuf_a, tbuf_b, sem_a, sem_b):
    wid = lax.axis_index("s") * _NC + lax.axis_index("c")
    base = wid * _BPW

    pltpu.sync_copy(user_hbm.at[pl.ds(base, _BPW)], uidx_v)
    pltpu.sync_copy(item_hbm.at[pl.ds(base, _BPW)], iidx_v)

    lane = lax.iota(jnp.int32, 16)

    def gather_table(tbl_hbm, idx_v, rows_out, tbuf, sem, tbuf2, sem2):
        def issue(c, buf, s):
            o = c * _CHUNK
            tcol = lax.shift_right_logical(idx_v[pl.ds(o, _CHUNK)], 7)
            for j in range(_CHUNK):
                col0 = pl.multiple_of(tcol[j] * 128, 128)
                pltpu.async_copy(
                    tbl_hbm.at[:, pl.ds(col0, 128)],
                    buf.at[pl.ds(j * 16, 16), :], s)

        def drain(buf, s):
            pltpu.make_async_copy(
                tbl_hbm.at[:, pl.ds(0, _CHUNK * 16)], buf, s).wait()

        def extract(c, buf):
            o = c * _CHUNK
            m16 = idx_v[pl.ds(o, _CHUNK)] & 127
            rowbase = lane * 16
            for d in range(NUM_FACTORS):
                vec = plsc.load_gather(buf, [rowbase + d, m16])
                rows_out[pl.ds(d * _BPW + o, 16)] = vec

        def pair(g, carry):
            c0 = g * 2
            issue(c0, tbuf, sem)

            @pl.when(g > 0)
            def _():
                drain(tbuf2, sem2)
                extract(c0 - 1, tbuf2)

            issue(c0 + 1, tbuf2, sem2)
            drain(tbuf, sem)
            extract(c0, tbuf)
            return carry

        lax.fori_loop(0, _NCHUNK // 2, pair, 0, unroll=False)
        drain(tbuf2, sem2)
        extract(_NCHUNK - 1, tbuf2)

    gather_table(tu_hbm, uidx_v, rows_u, tbuf_a, sem_a, tbuf_b, sem_b)
    gather_table(tv_hbm, iidx_v, rows_v, tbuf_a, sem_a, tbuf_b, sem_b)

    def step(g, carry):
        r0 = g * 16
        acc = jnp.zeros((16,), jnp.float32)
        for d in range(NUM_FACTORS):
            acc = acc + (rows_u[pl.ds(d * _BPW + r0, 16)]
                         * rows_v[pl.ds(d * _BPW + r0, 16)])
        out_v[pl.ds(r0, 16)] = acc
        return carry

    lax.fori_loop(0, _BPW // 16, step, 0, unroll=False)

    pltpu.sync_copy(out_v, out_hbm.at[pl.ds(base, _BPW)])


@jax.jit
def _mf_call(user, item, tu, tv):
    mesh = plsc.VectorSubcoreMesh(
        core_axis_name="c", subcore_axis_name="s",
        num_cores=_NC, num_subcores=_NS)
    return pl.kernel(
        _mf_body,
        out_type=jax.ShapeDtypeStruct((BATCH,), jnp.float32),
        mesh=mesh,
        compiler_params=pltpu.CompilerParams(
            needs_layout_passes=False, use_tc_tiling_on_sc=True),
        scratch_types=[
            pltpu.VMEM((_BPW,), jnp.int32),
            pltpu.VMEM((_BPW,), jnp.int32),
            pltpu.VMEM((_BPW * NUM_FACTORS,), jnp.float32),
            pltpu.VMEM((_BPW * NUM_FACTORS,), jnp.float32),
            pltpu.VMEM((_BPW,), jnp.float32),
            pltpu.VMEM((_CHUNK * 16, 128), jnp.float32),
            pltpu.VMEM((_CHUNK * 16, 128), jnp.float32),
            pltpu.SemaphoreType.DMA,
            pltpu.SemaphoreType.DMA,
        ],
    )(user, item, tu, tv)


def kernel(user, item, user_factors, item_factors):
    user = user.astype(jnp.int32)
    item = item.astype(jnp.int32)
    return _mf_call(user, item, user_factors.T, item_factors.T)


# speedup vs baseline: 6.8772x; 1.0570x over previous
import functools

import jax
import jax.numpy as jnp
from jax import lax
from jax.experimental import pallas as pl
from jax.experimental.pallas import tpu as pltpu
from jax.experimental.pallas import tpu_sc as plsc

NUM_FACTORS = 16
NUM_ROWS = 1000000
BATCH = 16384

_NC, _NS = 2, 16
_NW = _NC * _NS
_BPW = BATCH // _NW
_CHUNK = 16
_NCHUNK = _BPW // _CHUNK


def _mf_body(user_hbm, item_hbm, tu_hbm, tv_hbm, out_hbm,
             uidx_v, iidx_v, rows_u, rows_v, out_v, tbuf_a, tbuf_b, tbuf_c, sem_a, sem_b, sem_c):
    wid = lax.axis_index("s") * _NC + lax.axis_index("c")
    base = wid * _BPW

    pltpu.sync_copy(user_hbm.at[pl.ds(base, _BPW)], uidx_v)
    pltpu.sync_copy(item_hbm.at[pl.ds(base, _BPW)], iidx_v)

    lane = lax.iota(jnp.int32, 16)

    def gather_table(tbl_hbm, idx_v, rows_out, tbuf, sem, tbuf2, sem2, tbuf3, sem3):
        def issue(c, buf, s):
            o = c * _CHUNK
            tcol = lax.shift_right_logical(idx_v[pl.ds(o, _CHUNK)], 7)
            for j in range(_CHUNK):
                col0 = pl.multiple_of(tcol[j] * 128, 128)
                pltpu.async_copy(
                    tbl_hbm.at[:, pl.ds(col0, 128)],
                    buf.at[pl.ds(j * 16, 16), :], s)

        def drain(buf, s):
            pltpu.make_async_copy(
                tbl_hbm.at[:, pl.ds(0, _CHUNK * 16)], buf, s).wait()

        def extract(c, buf):
            o = c * _CHUNK
            m16 = idx_v[pl.ds(o, _CHUNK)] & 127
            rowbase = lane * 16
            for d in range(NUM_FACTORS):
                vec = plsc.load_gather(buf, [rowbase + d, m16])
                rows_out[pl.ds(d * _BPW + o, 16)] = vec

        def triple(g, carry):
            c0 = g * 3
            issue(c0, tbuf, sem)

            @pl.when(g > 0)
            def _():
                drain(tbuf2, sem2)
                extract(c0 - 2, tbuf2)

            issue(c0 + 1, tbuf2, sem2)

            @pl.when(g > 0)
            def _():
                drain(tbuf3, sem3)
                extract(c0 - 1, tbuf3)

            issue(c0 + 2, tbuf3, sem3)
            drain(tbuf, sem)
            extract(c0, tbuf)
            return carry

        nt = (_NCHUNK - 2) // 3
        lax.fori_loop(0, nt, triple, 0, unroll=False)
        c0 = nt * 3
        issue(c0, tbuf, sem)
        drain(tbuf2, sem2)
        extract(c0 - 2, tbuf2)
        issue(c0 + 1, tbuf2, sem2)
        drain(tbuf3, sem3)
        extract(c0 - 1, tbuf3)
        drain(tbuf, sem)
        extract(c0, tbuf)
        drain(tbuf2, sem2)
        extract(c0 + 1, tbuf2)

    gather_table(tu_hbm, uidx_v, rows_u, tbuf_a, sem_a, tbuf_b, sem_b, tbuf_c, sem_c)
    gather_table(tv_hbm, iidx_v, rows_v, tbuf_a, sem_a, tbuf_b, sem_b, tbuf_c, sem_c)

    def step(g, carry):
        r0 = g * 16
        acc = jnp.zeros((16,), jnp.float32)
        for d in range(NUM_FACTORS):
            acc = acc + (rows_u[pl.ds(d * _BPW + r0, 16)]
                         * rows_v[pl.ds(d * _BPW + r0, 16)])
        out_v[pl.ds(r0, 16)] = acc
        return carry

    lax.fori_loop(0, _BPW // 16, step, 0, unroll=False)

    pltpu.sync_copy(out_v, out_hbm.at[pl.ds(base, _BPW)])


@jax.jit
def _mf_call(user, item, tu, tv):
    mesh = plsc.VectorSubcoreMesh(
        core_axis_name="c", subcore_axis_name="s",
        num_cores=_NC, num_subcores=_NS)
    return pl.kernel(
        _mf_body,
        out_type=jax.ShapeDtypeStruct((BATCH,), jnp.float32),
        mesh=mesh,
        compiler_params=pltpu.CompilerParams(
            needs_layout_passes=False, use_tc_tiling_on_sc=True),
        scratch_types=[
            pltpu.VMEM((_BPW,), jnp.int32),
            pltpu.VMEM((_BPW,), jnp.int32),
            pltpu.VMEM((_BPW * NUM_FACTORS,), jnp.float32),
            pltpu.VMEM((_BPW * NUM_FACTORS,), jnp.float32),
            pltpu.VMEM((_BPW,), jnp.float32),
            pltpu.VMEM((_CHUNK * 16, 128), jnp.float32),
            pltpu.VMEM((_CHUNK * 16, 128), jnp.float32),
            pltpu.VMEM((_CHUNK * 16, 128), jnp.float32),
            pltpu.SemaphoreType.DMA,
            pltpu.SemaphoreType.DMA,
            pltpu.SemaphoreType.DMA,
        ],
    )(user, item, tu, tv)


def kernel(user, item, user_factors, item_factors):
    user = user.astype(jnp.int32)
    item = item.astype(jnp.int32)
    return _mf_call(user, item, user_factors.T, item_factors.T)


# speedup vs baseline: 7.0563x; 1.0260x over previous
import functools

import jax
import jax.numpy as jnp
from jax import lax
from jax.experimental import pallas as pl
from jax.experimental.pallas import tpu as pltpu
from jax.experimental.pallas import tpu_sc as plsc

NUM_FACTORS = 16
NUM_ROWS = 1000000
BATCH = 16384

_NC, _NS = 2, 16
_NW = _NC * _NS
_BPW = BATCH // _NW
_CHUNK = 16
_NCHUNK = _BPW // _CHUNK


def _mf_body(user_hbm, item_hbm, tu_hbm, tv_hbm, out_hbm,
             uidx_v, iidx_v, rows_u, rows_v, out_v, tbuf_a, tbuf_b, tbuf_c, sem_a, sem_b, sem_c):
    wid = lax.axis_index("s") * _NC + lax.axis_index("c")
    base = wid * _BPW

    pltpu.sync_copy(user_hbm.at[pl.ds(base, _BPW)], uidx_v)
    pltpu.sync_copy(item_hbm.at[pl.ds(base, _BPW)], iidx_v)

    lane = lax.iota(jnp.int32, 16)

    def gather_table(tbl_hbm, idx_v, rows_out, tbuf, sem, tbuf2, sem2, tbuf3, sem3):
        def issue(c, buf, s):
            o = c * _CHUNK
            tcol = lax.shift_right_logical(idx_v[pl.ds(o, _CHUNK)], 7)
            for j in range(_CHUNK):
                col0 = pl.multiple_of(tcol[j] * 128, 128)
                pltpu.async_copy(
                    tbl_hbm.at[pl.ds(0, 8), pl.ds(col0, 128)],
                    buf.at[pl.ds(j * 16, 8), :], s)
                pltpu.async_copy(
                    tbl_hbm.at[pl.ds(8, 8), pl.ds(col0, 128)],
                    buf.at[pl.ds(j * 16 + 8, 8), :], s)

        def drain(buf, s):
            pltpu.make_async_copy(
                tbl_hbm.at[:, pl.ds(0, _CHUNK * 16)], buf, s).wait()

        def extract(c, buf):
            o = c * _CHUNK
            m16 = idx_v[pl.ds(o, _CHUNK)] & 127
            rowbase = lane * 16
            for d in range(NUM_FACTORS):
                vec = plsc.load_gather(buf, [rowbase + d, m16])
                rows_out[pl.ds(d * _BPW + o, 16)] = vec

        def triple(g, carry):
            c0 = g * 3
            issue(c0, tbuf, sem)

            @pl.when(g > 0)
            def _():
                drain(tbuf2, sem2)
                extract(c0 - 2, tbuf2)

            issue(c0 + 1, tbuf2, sem2)

            @pl.when(g > 0)
            def _():
                drain(tbuf3, sem3)
                extract(c0 - 1, tbuf3)

            issue(c0 + 2, tbuf3, sem3)
            drain(tbuf, sem)
            extract(c0, tbuf)
            return carry

        nt = (_NCHUNK - 2) // 3
        lax.fori_loop(0, nt, triple, 0, unroll=False)
        c0 = nt * 3
        issue(c0, tbuf, sem)
        drain(tbuf2, sem2)
        extract(c0 - 2, tbuf2)
        issue(c0 + 1, tbuf2, sem2)
        drain(tbuf3, sem3)
        extract(c0 - 1, tbuf3)
        drain(tbuf, sem)
        extract(c0, tbuf)
        drain(tbuf2, sem2)
        extract(c0 + 1, tbuf2)

    gather_table(tu_hbm, uidx_v, rows_u, tbuf_a, sem_a, tbuf_b, sem_b, tbuf_c, sem_c)
    gather_table(tv_hbm, iidx_v, rows_v, tbuf_a, sem_a, tbuf_b, sem_b, tbuf_c, sem_c)

    def step(g, carry):
        r0 = g * 16
        acc = jnp.zeros((16,), jnp.float32)
        for d in range(NUM_FACTORS):
            acc = acc + (rows_u[pl.ds(d * _BPW + r0, 16)]
                         * rows_v[pl.ds(d * _BPW + r0, 16)])
        out_v[pl.ds(r0, 16)] = acc
        return carry

    lax.fori_loop(0, _BPW // 16, step, 0, unroll=False)

    pltpu.sync_copy(out_v, out_hbm.at[pl.ds(base, _BPW)])


@jax.jit
def _mf_call(user, item, tu, tv):
    mesh = plsc.VectorSubcoreMesh(
        core_axis_name="c", subcore_axis_name="s",
        num_cores=_NC, num_subcores=_NS)
    return pl.kernel(
        _mf_body,
        out_type=jax.ShapeDtypeStruct((BATCH,), jnp.float32),
        mesh=mesh,
        compiler_params=pltpu.CompilerParams(
            needs_layout_passes=False, use_tc_tiling_on_sc=True),
        scratch_types=[
            pltpu.VMEM((_BPW,), jnp.int32),
            pltpu.VMEM((_BPW,), jnp.int32),
            pltpu.VMEM((_BPW * NUM_FACTORS,), jnp.float32),
            pltpu.VMEM((_BPW * NUM_FACTORS,), jnp.float32),
            pltpu.VMEM((_BPW,), jnp.float32),
            pltpu.VMEM((_CHUNK * 16, 128), jnp.float32),
            pltpu.VMEM((_CHUNK * 16, 128), jnp.float32),
            pltpu.VMEM((_CHUNK * 16, 128), jnp.float32),
            pltpu.SemaphoreType.DMA,
            pltpu.SemaphoreType.DMA,
            pltpu.SemaphoreType.DMA,
        ],
    )(user, item, tu, tv)


def kernel(user, item, user_factors, item_factors):
    user = user.astype(jnp.int32)
    item = item.astype(jnp.int32)
    return _mf_call(user, item, user_factors.T, item_factors.T)
